# Initial kernel scaffold; baseline (speedup 1.0000x reference)
import jax
import jax.numpy as jnp
from jax.experimental import pallas as pl


def kernel(x, enc_W1, enc_b1, enc_W2, enc_b2, lip_W, lip_b, lip_ci, codebook, dec_W1, dec_b1, dec_W2, dec_b2, out_W, out_b):
    raise NotImplementedError("write your pallas kernel here")


# speedup vs baseline: 8.3143x; 8.3143x over previous
import jax
import jax.numpy as jnp
from jax.experimental import pallas as pl

_B, _F, _HID, _LAT, _K = 2048, 512, 128, 64, 1024
_BLK = 256
_PREC = jax.lax.Precision.DEFAULT


def _gelu(v):
    return 0.5 * v * (1.0 + jax.lax.erf(v * jnp.float32(0.7071067811865476)))


def _mm_t(a, b):
    return jax.lax.dot_general(a, b, (((1,), (1,)), ((), ())),
                               precision=_PREC,
                               preferred_element_type=jnp.float32)


def _mm(a, b):
    return jax.lax.dot_general(a, b, (((1,), (0,)), ((), ())),
                               precision=_PREC,
                               preferred_element_type=jnp.float32)


def _fused_kernel(x_ref, w1_ref, b1_ref, w2_ref, b2_ref, lw_ref, lb_ref,
                  lci_ref, cb_ref, dw1_ref, db1_ref, dw2_ref, db2_ref,
                  ow_ref, ob_ref, zq_ref, loss_ref):
    i = pl.program_id(0)
    x = x_ref[...]
    h = _gelu(_mm_t(x, w1_ref[...]) + b1_ref[...])
    h = _gelu(_mm_t(h, w2_ref[...]) + b2_ref[...])
    lw = lw_ref[...]
    absrowsum = jnp.sum(jnp.abs(lw), axis=1, keepdims=True)
    scale = jnp.minimum(jnp.float32(1.0),
                        jax.nn.softplus(lci_ref[...]) / absrowsum)
    wn = lw * scale
    z_e = jax.nn.sigmoid(_mm_t(h, wn) + lb_ref[...])
    cb = cb_ref[...]
    cnorm = _mm_t(jnp.ones((1, _LAT), jnp.float32), cb * cb)
    scores = cnorm - 2.0 * _mm_t(z_e, cb)
    mins = jnp.min(scores, axis=1, keepdims=True)
    iota = jax.lax.broadcasted_iota(jnp.int32, (_BLK, _K), 1)
    idx = jnp.min(jnp.where(scores == mins, iota, _K), axis=1, keepdims=True)
    onehot = (iota == idx).astype(jnp.float32)
    z_q = _mm(onehot, cb)
    zq_ref[...] = z_q
    r = _gelu(_mm_t(z_q, dw1_ref[...]) + db1_ref[...])
    r = _gelu(_mm_t(r, dw2_ref[...]) + db2_ref[...])
    xr = _mm_t(r, ow_ref[...]) + ob_ref[...]
    d = xr - x
    zd = z_q - z_e
    part = (jnp.sum(d * d) / jnp.float32(_B * _F)
            + 0.5 * jnp.sum(zd * zd) / jnp.float32(_B * _LAT))

    @pl.when(i == 0)
    def _init():
        loss_ref[...] = jnp.zeros_like(loss_ref)

    loss_ref[...] += part.reshape(1, 1)


@jax.jit
def kernel(x, enc_W1, enc_b1, enc_W2, enc_b2, lip_W, lip_b, lip_ci, codebook,
           dec_W1, dec_b1, dec_W2, dec_b2, out_W, out_b):
    full = lambda shape: pl.BlockSpec(shape, lambda i: (0, 0))
    z_q, loss = pl.pallas_call(
        _fused_kernel,
        grid=(_B // _BLK,),
        in_specs=[
            pl.BlockSpec((_BLK, _F), lambda i: (i, 0)),
            full((64, _F)), full((1, 64)),
            full((_HID, 64)), full((1, _HID)),
            full((_LAT, _HID)), full((1, _LAT)), full((_LAT, 1)),
            full((_K, _LAT)),
            full((64, _LAT)), full((1, 64)),
            full((_HID, 64)), full((1, _HID)),
            full((_F, _HID)), full((1, _F)),
        ],
        out_specs=[
            pl.BlockSpec((_BLK, _LAT), lambda i: (i, 0)),
            pl.BlockSpec((1, 1), lambda i: (0, 0)),
        ],
        out_shape=[
            jax.ShapeDtypeStruct((_B, _LAT), jnp.float32),
            jax.ShapeDtypeStruct((1, 1), jnp.float32),
        ],
    )(x, enc_W1, enc_b1.reshape(1, -1), enc_W2, enc_b2.reshape(1, -1),
      lip_W, lip_b.reshape(1, -1), lip_ci.reshape(-1, 1), codebook,
      dec_W1, dec_b1.reshape(1, -1), dec_W2, dec_b2.reshape(1, -1),
      out_W, out_b.reshape(1, -1))
    return z_q, loss[0, 0]


# speedup vs baseline: 10.0966x; 1.2144x over previous
import jax
import jax.numpy as jnp
from jax.experimental import pallas as pl

_B, _F, _HID, _LAT, _K = 2048, 512, 128, 64, 1024
_BLK = 512
_PREC = jax.lax.Precision.DEFAULT


def _gelu(v):
    return 0.5 * v * (1.0 + jax.lax.erf(v * jnp.float32(0.7071067811865476)))


def _mm_t(a, b):
    return jax.lax.dot_general(a, b, (((1,), (1,)), ((), ())),
                               precision=_PREC,
                               preferred_element_type=jnp.float32)


def _mm(a, b):
    return jax.lax.dot_general(a, b, (((1,), (0,)), ((), ())),
                               precision=_PREC,
                               preferred_element_type=jnp.float32)


def _fused_kernel(x_ref, w1_ref, b1_ref, w2_ref, b2_ref, lw_ref, lb_ref,
                  lci_ref, cb_ref, dw1_ref, db1_ref, dw2_ref, db2_ref,
                  ow_ref, ob_ref, zq_ref, loss_ref):
    i = pl.program_id(0)
    x = x_ref[...]
    h = _gelu(_mm_t(x, w1_ref[...]) + b1_ref[...])
    h = _gelu(_mm_t(h, w2_ref[...]) + b2_ref[...])
    lw = lw_ref[...]
    absrowsum = jnp.sum(jnp.abs(lw), axis=1, keepdims=True)
    scale = jnp.minimum(jnp.float32(1.0),
                        jax.nn.softplus(lci_ref[...]) / absrowsum)
    wn = lw * scale
    z_e = jax.nn.sigmoid(_mm_t(h, wn) + lb_ref[...])
    cb = cb_ref[...]
    cnorm = _mm_t(jnp.ones((1, _LAT), jnp.float32), cb * cb)
    scores = cnorm - 2.0 * _mm_t(z_e, cb)
    mins = jnp.min(scores, axis=1, keepdims=True)
    iota = jax.lax.broadcasted_iota(jnp.int32, (_BLK, _K), 1)
    idx = jnp.min(jnp.where(scores == mins, iota, _K), axis=1, keepdims=True)
    onehot = (iota == idx).astype(jnp.float32)
    z_q = _mm(onehot, cb)
    zq_ref[...] = z_q
    r = _gelu(_mm_t(z_q, dw1_ref[...]) + db1_ref[...])
    r = _gelu(_mm_t(r, dw2_ref[...]) + db2_ref[...])
    xr = _mm_t(r, ow_ref[...]) + ob_ref[...]
    d = xr - x
    zd = z_q - z_e
    part = (jnp.sum(d * d) / jnp.float32(_B * _F)
            + 0.5 * jnp.sum(zd * zd) / jnp.float32(_B * _LAT))

    @pl.when(i == 0)
    def _init():
        loss_ref[...] = jnp.zeros_like(loss_ref)

    loss_ref[...] += part.reshape(1, 1)


@jax.jit
def kernel(x, enc_W1, enc_b1, enc_W2, enc_b2, lip_W, lip_b, lip_ci, codebook,
           dec_W1, dec_b1, dec_W2, dec_b2, out_W, out_b):
    full = lambda shape: pl.BlockSpec(shape, lambda i: (0, 0))
    z_q, loss = pl.pallas_call(
        _fused_kernel,
        grid=(_B // _BLK,),
        in_specs=[
            pl.BlockSpec((_BLK, _F), lambda i: (i, 0)),
            full((64, _F)), full((1, 64)),
            full((_HID, 64)), full((1, _HID)),
            full((_LAT, _HID)), full((1, _LAT)), full((_LAT, 1)),
            full((_K, _LAT)),
            full((64, _LAT)), full((1, 64)),
            full((_HID, 64)), full((1, _HID)),
            full((_F, _HID)), full((1, _F)),
        ],
        out_specs=[
            pl.BlockSpec((_BLK, _LAT), lambda i: (i, 0)),
            pl.BlockSpec((1, 1), lambda i: (0, 0)),
        ],
        out_shape=[
            jax.ShapeDtypeStruct((_B, _LAT), jnp.float32),
            jax.ShapeDtypeStruct((1, 1), jnp.float32),
        ],
    )(x, enc_W1, enc_b1.reshape(1, -1), enc_W2, enc_b2.reshape(1, -1),
      lip_W, lip_b.reshape(1, -1), lip_ci.reshape(-1, 1), codebook,
      dec_W1, dec_b1.reshape(1, -1), dec_W2, dec_b2.reshape(1, -1),
      out_W, out_b.reshape(1, -1))
    return z_q, loss[0, 0]
